# baseline (speedup 1.0000x reference)
import functools

import jax
import jax.numpy as jnp
from jax import lax
from jax.experimental import pallas as pl
from jax.experimental.pallas import tpu as pltpu
from jax.experimental.pallas import tpu_sc as plsc

NC = 2
NS = 16
NW = NC * NS

CH = 128
_NBUF = 4

_BLKV = 8064
_NREG = 31
_HQ = _BLKV * _NREG


def _tr(t, eye):
    return lax.dot_general(t, eye, (((0,), (0,)), ((), ())),
                           preferred_element_type=jnp.float32)


def _pack(t, eye):
    y = _tr(t, eye)
    xi = lax.bitcast_convert_type(y, jnp.int32)
    xb = lax.shift_right_logical(xi + 0x7FFF + ((xi >> 16) & 1), 16)
    return xb[:, :32] | (xb[:, 32:] << 16)


def _repack_body(t0_ref, t1_ref, t2_ref, t3_ref, eye_ref, o_ref):
    i = pl.program_id(0)
    eye = eye_ref[...]
    trefs = (t0_ref, t1_ref, t2_ref, t3_ref)

    @pl.when(i < _NREG)
    def _():
        for q in range(4):
            o_ref[:, 32 * q:32 * (q + 1)] = _pack(trefs[q][...], eye)

    @pl.when(i == _NREG)
    def _():
        t = t0_ref[...]
        for q in range(4):
            o_ref[:16, 32 * q:32 * (q + 1)] = _pack(t[:, 16 * q:16 * (q + 1)],
                                                    eye)


def _repack(tT):
    D, V = tT.shape
    eye = jnp.eye(D, dtype=jnp.float32)

    def mk_map(q):
        return lambda i: (0, jnp.where(i == _NREG,
                                       4 * _NREG if q == 0 else 0,
                                       i + q * _NREG))

    return pl.pallas_call(
        _repack_body,
        grid=(_NREG + 1,),
        in_specs=[pl.BlockSpec((D, _BLKV), mk_map(q)) for q in range(4)]
        + [pl.BlockSpec((D, D), lambda i: (0, 0))],
        out_specs=pl.BlockSpec((_BLKV, 128), lambda i: (i, 0)),
        out_shape=jax.ShapeDtypeStruct((V // 4, 128), jnp.int32),
    )(tT, tT, tT, tT, eye)


def _make_gather(B, DW):
    b_per_w = B // NW
    nch = b_per_w // CH
    mesh = plsc.VectorSubcoreMesh(core_axis_name="c", subcore_axis_name="s")

    @functools.partial(
        pl.kernel,
        mesh=mesh,
        compiler_params=pltpu.CompilerParams(use_tc_tiling_on_sc=False),
        out_type=jax.ShapeDtypeStruct((B, DW), jnp.int32),
        scratch_types=[
            pltpu.VMEM((nch, CH), jnp.int32),
        ]
        + [pltpu.VMEM((CH, DW), jnp.int32) for _ in range(_NBUF)]
        + [pltpu.SemaphoreType.DMA for _ in range(_NBUF)],
    )
    def gather_k(idx_hbm, table_hbm, out_hbm, idx_v, *bufs_sems):
        rows = bufs_sems[:_NBUF]
        sems = bufs_sems[_NBUF:]
        wid = lax.axis_index("s") * NC + lax.axis_index("c")
        base = wid * b_per_w
        pltpu.sync_copy(idx_hbm.at[wid], idx_v)
        for b in range(_NBUF):
            pltpu.async_copy(table_hbm.at[idx_v.at[b]], rows[b], sems[b])

        def step(j, rows_b, sem_b):
            pltpu.make_async_copy(table_hbm.at[idx_v.at[j]], rows_b,
                                  sem_b).wait()
            pltpu.sync_copy(rows_b, out_hbm.at[pl.ds(base + j * CH, CH)])

            @pl.when(j + _NBUF < nch)
            def _():
                pltpu.async_copy(table_hbm.at[idx_v.at[j + _NBUF]], rows_b,
                                 sem_b)

        def body(jg, carry):
            for b in range(_NBUF):
                step(_NBUF * jg + b, rows[b], sems[b])
            return carry

        lax.fori_loop(0, nch // _NBUF, body, 0)

    return gather_k


def _mm_body(e_ref, wlo_ref, whi_ref, o_ref):
    w = e_ref[0]
    flo = lax.bitcast_convert_type(w << 16, jnp.float32)
    fhi = lax.bitcast_convert_type(w & jnp.int32(-65536), jnp.float32)
    y = (lax.dot_general(wlo_ref[...], flo, (((1,), (1,)), ((), ())),
                         preferred_element_type=jnp.float32)
         + lax.dot_general(whi_ref[...], fhi, (((1,), (1,)), ((), ())),
                           preferred_element_type=jnp.float32))
    r = w.shape[0]
    for q in range(4):
        o_ref[0, :, r * q:r * (q + 1)] = y[64 * q:64 * (q + 1), :]


def _matmul(e3, wlo, whi, L, Bt, O):
    rows = Bt // 4
    return pl.pallas_call(
        _mm_body,
        grid=(L,),
        in_specs=[
            pl.BlockSpec((1, rows, 128), lambda i: (i, 0, 0)),
            pl.BlockSpec(wlo.shape, lambda i: (0, 0)),
            pl.BlockSpec(whi.shape, lambda i: (0, 0)),
        ],
        out_specs=pl.BlockSpec((1, O, Bt), lambda i: (i, 0, 0)),
        out_shape=jax.ShapeDtypeStruct((L, O, Bt), jnp.float32),
    )(e3, wlo, whi)


def kernel(x, emb_table, fc_w):
    Bt, L = x.shape
    B = Bt * L
    O = fc_w.shape[0]

    tT = jnp.transpose(emb_table)
    t2c = _repack(tT)
    t_rm = t2c.reshape(emb_table.shape[0], 32)

    xT = jnp.transpose(x)
    xp = jnp.transpose(xT.reshape(L, 4, Bt // 4), (0, 2, 1))
    v = xp
    s = jnp.where(v < _HQ, 4 * v,
        jnp.where(v < 2 * _HQ, 4 * (v - _HQ) + 1,
        jnp.where(v < 3 * _HQ, 4 * (v - 2 * _HQ) + 2,
        jnp.where(v < 4 * _HQ, 4 * (v - 3 * _HQ) + 3,
        jnp.where(v < 999952, 4 * v - 2999808,
        jnp.where(v < 999968, 4 * v - 2999871,
        jnp.where(v < 999984, 4 * v - 2999934,
                  4 * v - 2999997)))))))
    idx = s.reshape(NW, (B // NW) // CH, CH).astype(jnp.int32)

    e = _make_gather(B, 32)(idx, t_rm)
    e3 = e.reshape(L, Bt // 4, 128)

    D = emb_table.shape[1]
    wlo = jnp.zeros((4 * O, 128), jnp.float32)
    whi = jnp.zeros((4 * O, 128), jnp.float32)
    for q in range(4):
        wlo = wlo.at[64 * q:64 * (q + 1), 32 * q:32 * (q + 1)].set(
            fc_w[:, :D // 2])
        whi = whi.at[64 * q:64 * (q + 1), 32 * q:32 * (q + 1)].set(
            fc_w[:, D // 2:])
    out_t = _matmul(e3, wlo, whi, L, Bt, O)
    return jnp.transpose(out_t, (2, 0, 1))


# speedup vs baseline: 1.6756x; 1.6756x over previous
import functools

import jax
import jax.numpy as jnp
from jax import lax
from jax.experimental import pallas as pl
from jax.experimental.pallas import tpu as pltpu
from jax.experimental.pallas import tpu_sc as plsc

NC = 2
NS = 16
NW = NC * NS

CH = 128
_NBUF = 8


_BLKV = 8064
_NREG = 62
_H = _BLKV * _NREG


def _tr(t, eye):
    return lax.dot_general(t, eye, (((0,), (0,)), ((), ())),
                           preferred_element_type=jnp.float32)


def _repack_body(t0_ref, t1_ref, eye_ref, o_ref):
    i = pl.program_id(0)
    D = t0_ref.shape[0]
    eye = eye_ref[...]

    @pl.when(i < _NREG)
    def _():
        o_ref[:, :D] = _tr(t0_ref[...], eye)
        o_ref[:, D:] = _tr(t1_ref[...], eye)

    @pl.when(i == _NREG)
    def _():
        t = t0_ref[...]
        o_ref[:32, :D] = _tr(t[:, :32], eye)
        o_ref[:32, D:] = _tr(t[:, 32:64], eye)


def _repack(tT):
    D, V = tT.shape
    eye = jnp.eye(D, dtype=jnp.float32)
    return pl.pallas_call(
        _repack_body,
        grid=(_NREG + 1,),
        in_specs=[
            pl.BlockSpec((D, _BLKV),
                         lambda i: (0, jnp.where(i == _NREG, 2 * _NREG, i))),
            pl.BlockSpec((D, _BLKV),
                         lambda i: (0, jnp.where(i == _NREG, 0, i + _NREG))),
            pl.BlockSpec((D, D), lambda i: (0, 0)),
        ],
        out_specs=pl.BlockSpec((_BLKV, 2 * D), lambda i: (i, 0)),
        out_shape=jax.ShapeDtypeStruct((V // 2, 2 * D), jnp.float32),
    )(tT, tT, eye)


def _make_gather(B, D):
    b_per_w = B // NW
    nch = b_per_w // CH
    mesh = plsc.VectorSubcoreMesh(core_axis_name="c", subcore_axis_name="s")

    @functools.partial(
        pl.kernel,
        mesh=mesh,
        compiler_params=pltpu.CompilerParams(use_tc_tiling_on_sc=False),
        out_type=jax.ShapeDtypeStruct((B, D), jnp.float32),
        scratch_types=[
            pltpu.VMEM((nch, CH), jnp.int32),
        ]
        + [pltpu.VMEM((CH, D), jnp.float32) for _ in range(_NBUF)]
        + [pltpu.SemaphoreType.DMA for _ in range(_NBUF)],
    )
    def gather_k(idx_hbm, table_hbm, out_hbm, idx_v, *bufs_sems):
        rows = bufs_sems[:_NBUF]
        sems = bufs_sems[_NBUF:]
        wid = lax.axis_index("s") * NC + lax.axis_index("c")
        base = wid * b_per_w
        pltpu.sync_copy(idx_hbm.at[wid], idx_v)
        for b in range(_NBUF):
            pltpu.async_copy(table_hbm.at[idx_v.at[b]], rows[b], sems[b])

        def step(j, rows_b, sem_b):
            pltpu.make_async_copy(table_hbm.at[idx_v.at[j]], rows_b,
                                  sem_b).wait()
            pltpu.sync_copy(rows_b, out_hbm.at[pl.ds(base + j * CH, CH)])

            @pl.when(j + _NBUF < nch)
            def _():
                pltpu.async_copy(table_hbm.at[idx_v.at[j + _NBUF]], rows_b,
                                 sem_b)

        def body(jg, carry):
            for b in range(_NBUF):
                step(_NBUF * jg + b, rows[b], sems[b])
            return carry

        lax.fori_loop(0, nch // _NBUF, body, 0)

    return gather_k


def _mm_compute(e_ref, w_ref, o_ref):
    y = lax.dot_general(
        w_ref[...], e_ref[0],
        (((1,), (1,)), ((), ())),
        preferred_element_type=jnp.float32,
    )
    half = y.shape[1]
    O = o_ref.shape[1]
    o_ref[0, :, :half] = y[:O, :]
    o_ref[0, :, half:] = y[O:, :]


def _matmul(ep3, w2, L, Bt, O):
    half = Bt // 2
    return pl.pallas_call(
        _mm_compute,
        grid=(L,),
        in_specs=[
            pl.BlockSpec((1, half, w2.shape[0]), lambda i: (i, 0, 0)),
            pl.BlockSpec(w2.shape, lambda i: (0, 0)),
        ],
        out_specs=pl.BlockSpec((1, O, Bt), lambda i: (i, 0, 0)),
        out_shape=jax.ShapeDtypeStruct((L, O, Bt), jnp.float32),
    )(ep3, w2)


def kernel(x, emb_table, fc_w):
    Bt, L = x.shape
    B = Bt * L
    D = emb_table.shape[1]
    O = fc_w.shape[0]

    tT = jnp.transpose(emb_table)
    t2d = _repack(tT)
    t_rm = t2d.reshape(emb_table.shape)

    xT = jnp.transpose(x)
    xp = jnp.transpose(xT.reshape(L, 2, Bt // 2), (0, 2, 1))
    xp = jnp.where(
        xp < _H, 2 * xp,
        jnp.where(xp < 2 * _H, 2 * (xp - _H) + 1,
                  jnp.where(xp < 2 * _H + 32, 2 * xp - 2 * _H,
                            2 * xp - 999999)))
    idx = xp.reshape(NW, (B // NW) // CH, CH).astype(jnp.int32)

    e = _make_gather(B, D)(idx, t_rm)
    ep3 = e.reshape(L, Bt // 2, 2 * D)

    w2 = jnp.zeros((2 * D, 2 * O), jnp.float32)
    w2 = w2.at[:O, :D].set(fc_w).at[O:, D:].set(fc_w)
    out_t = _matmul(ep3, w2, L, Bt, O)
    return jnp.transpose(out_t, (2, 0, 1))


# speedup vs baseline: 1.6807x; 1.0030x over previous
import functools

import jax
import jax.numpy as jnp
from jax import lax
from jax.experimental import pallas as pl
from jax.experimental.pallas import tpu as pltpu
from jax.experimental.pallas import tpu_sc as plsc

NC = 2
NS = 16
NW = NC * NS

CH = 128
_NBUF = 4


_BLKV = 8064
_NREG = 62
_H = _BLKV * _NREG


def _tr(t, eye):
    return lax.dot_general(t, eye, (((0,), (0,)), ((), ())),
                           preferred_element_type=jnp.float32)


def _repack_body(t0_ref, t1_ref, eye_ref, o_ref):
    i = pl.program_id(0)
    D = t0_ref.shape[0]
    eye = eye_ref[...]

    @pl.when(i < _NREG)
    def _():
        o_ref[:, :D] = _tr(t0_ref[...], eye)
        o_ref[:, D:] = _tr(t1_ref[...], eye)

    @pl.when(i == _NREG)
    def _():
        t = t0_ref[...]
        o_ref[:32, :D] = _tr(t[:, :32], eye)
        o_ref[:32, D:] = _tr(t[:, 32:64], eye)


def _repack(tT):
    D, V = tT.shape
    eye = jnp.eye(D, dtype=jnp.float32)
    return pl.pallas_call(
        _repack_body,
        grid=(_NREG + 1,),
        in_specs=[
            pl.BlockSpec((D, _BLKV),
                         lambda i: (0, jnp.where(i == _NREG, 2 * _NREG, i))),
            pl.BlockSpec((D, _BLKV),
                         lambda i: (0, jnp.where(i == _NREG, 0, i + _NREG))),
            pl.BlockSpec((D, D), lambda i: (0, 0)),
        ],
        out_specs=pl.BlockSpec((_BLKV, 2 * D), lambda i: (i, 0)),
        out_shape=jax.ShapeDtypeStruct((V // 2, 2 * D), jnp.float32),
    )(tT, tT, eye)


def _make_gather(B, D):
    b_per_w = B // NW
    nch = b_per_w // CH
    mesh = plsc.VectorSubcoreMesh(core_axis_name="c", subcore_axis_name="s")

    @functools.partial(
        pl.kernel,
        mesh=mesh,
        compiler_params=pltpu.CompilerParams(use_tc_tiling_on_sc=False),
        out_type=jax.ShapeDtypeStruct((B, D), jnp.float32),
        scratch_types=[
            pltpu.VMEM((nch, CH), jnp.int32),
        ]
        + [pltpu.VMEM((CH, D), jnp.float32) for _ in range(_NBUF)]
        + [pltpu.SemaphoreType.DMA for _ in range(_NBUF)],
    )
    def gather_k(idx_hbm, table_hbm, out_hbm, idx_v, *bufs_sems):
        rows = bufs_sems[:_NBUF]
        sems = bufs_sems[_NBUF:]
        wid = lax.axis_index("s") * NC + lax.axis_index("c")
        base = wid * b_per_w
        pltpu.sync_copy(idx_hbm.at[wid], idx_v)
        for b in range(_NBUF):
            pltpu.async_copy(table_hbm.at[idx_v.at[b]], rows[b], sems[b])

        def step(j, rows_b, sem_b):
            pltpu.make_async_copy(table_hbm.at[idx_v.at[j]], rows_b,
                                  sem_b).wait()
            pltpu.sync_copy(rows_b, out_hbm.at[pl.ds(base + j * CH, CH)])

            @pl.when(j + _NBUF < nch)
            def _():
                pltpu.async_copy(table_hbm.at[idx_v.at[j + _NBUF]], rows_b,
                                 sem_b)

        def body(jg, carry):
            for b in range(_NBUF):
                step(_NBUF * jg + b, rows[b], sems[b])
            return carry

        lax.fori_loop(0, nch // _NBUF, body, 0)

    return gather_k


def _mm_compute(e_ref, w_ref, o_ref):
    y = lax.dot_general(
        w_ref[...], e_ref[0],
        (((1,), (1,)), ((), ())),
        preferred_element_type=jnp.float32,
    )
    half = y.shape[1]
    O = o_ref.shape[1]
    o_ref[0, :, :half] = y[:O, :]
    o_ref[0, :, half:] = y[O:, :]


def _matmul(ep3, w2, L, Bt, O):
    half = Bt // 2
    return pl.pallas_call(
        _mm_compute,
        grid=(L,),
        in_specs=[
            pl.BlockSpec((1, half, w2.shape[0]), lambda i: (i, 0, 0)),
            pl.BlockSpec(w2.shape, lambda i: (0, 0)),
        ],
        out_specs=pl.BlockSpec((1, O, Bt), lambda i: (i, 0, 0)),
        out_shape=jax.ShapeDtypeStruct((L, O, Bt), jnp.float32),
    )(ep3, w2)


def kernel(x, emb_table, fc_w):
    Bt, L = x.shape
    B = Bt * L
    D = emb_table.shape[1]
    O = fc_w.shape[0]

    tT = jnp.transpose(emb_table)
    t2d = _repack(tT)
    t_rm = t2d.reshape(emb_table.shape)

    xT = jnp.transpose(x)
    xp = jnp.transpose(xT.reshape(L, 2, Bt // 2), (0, 2, 1))
    xp = jnp.where(
        xp < _H, 2 * xp,
        jnp.where(xp < 2 * _H, 2 * (xp - _H) + 1,
                  jnp.where(xp < 2 * _H + 32, 2 * xp - 2 * _H,
                            2 * xp - 999999)))
    idx = xp.reshape(NW, (B // NW) // CH, CH).astype(jnp.int32)

    e = _make_gather(B, D)(idx, t_rm)
    ep3 = e.reshape(L, Bt // 2, 2 * D)

    w2 = jnp.zeros((2 * D, 2 * O), jnp.float32)
    w2 = w2.at[:O, :D].set(fc_w).at[O:, D:].set(fc_w)
    out_t = _matmul(ep3, w2, L, Bt, O)
    return jnp.transpose(out_t, (2, 0, 1))
